# baseline (speedup 1.0000x reference)
import functools

import jax
import jax.numpy as jnp
from jax import lax
from jax.experimental import pallas as pl
from jax.experimental.pallas import tpu as pltpu
from jax.experimental.pallas import tpu_sc as plsc

E = 8
K = 2
D = 1024
H = 2048
N = 2048
NP = N * K
BLK = 256
NBR = NP // BLK + E
NBS = N // BLK
NB = NBR + NBS
XS_ROWS = NBR * BLK
CHUNK = 128

NC, NS, L = 2, 16, 16
NW = NC * NS
TPT = N // NW


def _router_body(x_ref, rw_ref, pos_ref, w16_ref, eb_ref, nbu_ref,
                 rzl_ref, lb_ref):
    xv = x_ref[...]
    rw = rw_ref[...]
    logits = lax.dot_general(xv, rw, (((1,), (1,)), ((), ())),
                             preferred_element_type=jnp.float32)
    n = logits.shape[0]
    iota_e = lax.broadcasted_iota(jnp.int32, (n, E), 1)
    m1 = jnp.max(logits, axis=1, keepdims=True)
    i1 = jnp.min(jnp.where(logits == m1, iota_e, E), axis=1, keepdims=True)
    l2 = jnp.where(iota_e == i1, -jnp.inf, logits)
    m2 = jnp.max(l2, axis=1, keepdims=True)
    i2 = jnp.min(jnp.where(l2 == m2, iota_e, E), axis=1, keepdims=True)
    s = jnp.exp(m2 - m1)
    w1 = 1.0 / (1.0 + s)
    w2 = s * w1
    wp = jnp.concatenate([w1, w2], axis=0)
    w16_ref[...] = jnp.broadcast_to(wp, (NP, 128))
    ez = jnp.exp(logits - m1)
    denom = jnp.sum(ez, axis=1, keepdims=True)
    z = m1 + jnp.log(denom)
    rzl_ref[...] = jnp.reshape(jnp.sum(z * z) * (1e-4 / n), (1, 1))
    p_sum = jnp.sum(ez / denom, axis=0, keepdims=True)
    mask = jnp.logical_or(iota_e == i1, iota_e == i2)
    f_sum = jnp.sum(mask.astype(jnp.float32), axis=0, keepdims=True)
    lb_ref[...] = jnp.reshape(jnp.sum(p_sum * f_sum) * (float(E) / (n * n)),
                              (1, 1))

    ep = jnp.concatenate([i1, i2], axis=0)
    iota_pe = lax.broadcasted_iota(jnp.int32, (NP, E), 1)
    oh = (iota_pe == ep).astype(jnp.float32)
    counts = jnp.sum(oh, axis=0, keepdims=True)
    cp = jnp.floor((counts + (BLK - 1)) * (1.0 / BLK)) * BLK
    su = (lax.broadcasted_iota(jnp.int32, (E, E), 0)
          < lax.broadcasted_iota(jnp.int32, (E, E), 1)).astype(jnp.float32)
    offp = lax.dot_general(cp, su, (((1,), (0,)), ((), ())),
                           preferred_element_type=jnp.float32)
    nbu = jnp.sum(cp) * (1.0 / BLK)
    nbu_ref[...] = jnp.reshape(nbu, (1, 1)).astype(jnp.int32)

    tri = (lax.broadcasted_iota(jnp.int32, (CHUNK, CHUNK), 0)
           > lax.broadcasted_iota(jnp.int32, (CHUNK, CHUNK), 1)
           ).astype(jnp.float32)
    prefix = jnp.zeros((1, E), jnp.float32)
    pos_parts = []
    for c in range(NP // CHUNK):
        oh_c = oh[c * CHUNK:(c + 1) * CHUNK]
        rc = lax.dot_general(tri, oh_c, (((1,), (0,)), ((), ())),
                             preferred_element_type=jnp.float32) + prefix
        pos_c = jnp.sum(oh_c * (rc + offp), axis=1, keepdims=True)
        pos_parts.append(pos_c)
        prefix = prefix + jnp.sum(oh_c, axis=0, keepdims=True)
    pos = jnp.concatenate(pos_parts, axis=0)
    pos_ref[...] = pos.astype(jnp.int32)

    bi = lax.broadcasted_iota(jnp.int32, (NB, 1), 0).astype(jnp.float32)
    rowpos = jnp.minimum(bi * BLK, (nbu - 1.0) * BLK)
    le = (jnp.broadcast_to(offp, (NB, E)) <= rowpos).astype(jnp.float32)
    eb = jnp.sum(le, axis=1, keepdims=True) - 1.0
    eb = jnp.where(bi >= float(NBR), float(E), eb)
    eb_ref[...] = eb.astype(jnp.int32)


def _dispatch_body(x_hbm, w16_hbm, pos1_hbm, pos2_hbm, xs_hbm, ws_hbm,
                   rows_v, w1r_v, w2r_v, idx1_v, idx2_v, sem):
    wid = lax.axis_index("s") * NC + lax.axis_index("c")
    base = wid * TPT
    pltpu.sync_copy(x_hbm.at[pl.ds(base, TPT)], rows_v)
    pltpu.sync_copy(pos1_hbm.at[pl.ds(base, TPT)], idx1_v)
    pltpu.sync_copy(pos2_hbm.at[pl.ds(base, TPT)], idx2_v)
    pltpu.sync_copy(w16_hbm.at[pl.ds(base, TPT)], w1r_v)
    pltpu.sync_copy(w16_hbm.at[pl.ds(N + base, TPT)], w2r_v)
    pltpu.async_copy(rows_v, xs_hbm.at[idx1_v], sem).wait()
    pltpu.async_copy(rows_v, xs_hbm.at[idx2_v], sem).wait()
    pltpu.async_copy(w1r_v, ws_hbm.at[idx1_v], sem).wait()
    pltpu.async_copy(w2r_v, ws_hbm.at[idx2_v], sem).wait()


def _gemm_block(x_ref, gw_ref, uw_ref, dw_ref, wcol):
    xb = x_ref[...]
    gw = gw_ref[...]
    uw = uw_ref[...]
    dw = dw_ref[...]
    if gw.ndim == 3:
        gw, uw, dw = gw[0], uw[0], dw[0]
    g = lax.dot_general(xb, gw, (((1,), (1,)), ((), ())),
                        preferred_element_type=jnp.float32)
    u = lax.dot_general(xb, uw, (((1,), (1,)), ((), ())),
                        preferred_element_type=jnp.float32)
    h = jax.nn.silu(g) * u
    if wcol is not None:
        h = h * wcol
    return lax.dot_general(h, dw, (((1,), (1,)), ((), ())),
                           preferred_element_type=jnp.float32)


def _gemm_routed_body(s_ref, xs_ref, ws_ref, gw_ref, uw_ref, dw_ref,
                      os_ref):
    b = pl.program_id(0)
    hb = pl.program_id(1)
    nbu = s_ref[NBR]

    @pl.when(b < nbu)
    def _():
        xb = xs_ref[...]
        g = lax.dot_general(xb, gw_ref[0], (((1,), (1,)), ((), ())),
                            preferred_element_type=jnp.float32)
        u = lax.dot_general(xb, uw_ref[0], (((1,), (1,)), ((), ())),
                            preferred_element_type=jnp.float32)
        h = jax.nn.silu(g) * u * ws_ref[...][:, 0:1]
        contrib = lax.dot_general(h, dw_ref[0], (((1,), (1,)), ((), ())),
                                  preferred_element_type=jnp.float32)

        @pl.when(hb == 0)
        def _():
            os_ref[...] = contrib

        @pl.when(hb != 0)
        def _():
            os_ref[...] += contrib


def _gemm_shared_body(xf_ref, gw_ref, uw_ref, dw_ref, sh_ref):
    sh_ref[...] = _gemm_block(xf_ref, gw_ref, uw_ref, dw_ref, None)


_CS = 16


def _combine_body(os_hbm, sh_hbm, pos1_hbm, pos2_hbm,
                  out_hbm, b1_v, b2_v, shb_v, outb_v, i1_v, i2_v, sem):
    wid = lax.axis_index("s") * NC + lax.axis_index("c")
    base = wid * TPT
    nvec = D // L
    for q in range(TPT // _CS):
        b0 = base + q * _CS
        pltpu.sync_copy(pos1_hbm.at[pl.ds(b0, _CS)], i1_v)
        pltpu.sync_copy(pos2_hbm.at[pl.ds(b0, _CS)], i2_v)
        pltpu.async_copy(os_hbm.at[i1_v], b1_v, sem).wait()
        pltpu.async_copy(os_hbm.at[i2_v], b2_v, sem).wait()
        pltpu.sync_copy(sh_hbm.at[pl.ds(b0, _CS)], shb_v)
        for j in range(_CS):
            def body(cv, carry, j=j):
                sl = pl.ds(cv * L, L)
                outb_v[j, sl] = b1_v[j, sl] + b2_v[j, sl] + shb_v[j, sl]
                return carry

            lax.fori_loop(0, nvec, body, 0)
        pltpu.sync_copy(outb_v, out_hbm.at[pl.ds(b0, _CS)])


def kernel(x, router_w, gate_w, up_w, down_w, shared_gate_w, shared_up_w,
           shared_down_w):
    b, s, d = x.shape
    flat = x.reshape(N, D)

    pos, w16, eb, nbu, rzl, lb = pl.pallas_call(
        _router_body,
        out_shape=(
            jax.ShapeDtypeStruct((NP, 1), jnp.int32),
            jax.ShapeDtypeStruct((NP, 128), jnp.float32),
            jax.ShapeDtypeStruct((NB, 1), jnp.int32),
            jax.ShapeDtypeStruct((1, 1), jnp.int32),
            jax.ShapeDtypeStruct((1, 1), jnp.float32),
            jax.ShapeDtypeStruct((1, 1), jnp.float32),
        ),
    )(flat, router_w)

    pos1 = pos[:N, 0]
    pos2 = pos[N:, 0]

    mesh = plsc.VectorSubcoreMesh(core_axis_name="c", subcore_axis_name="s",
                                  num_cores=NC, num_subcores=NS)

    dispatch = functools.partial(
        pl.kernel,
        out_type=(
            jax.ShapeDtypeStruct((XS_ROWS, D), jnp.float32),
            jax.ShapeDtypeStruct((XS_ROWS, 128), jnp.float32),
        ),
        mesh=mesh,
        scratch_types=[
            pltpu.VMEM((TPT, D), jnp.float32),
            pltpu.VMEM((TPT, 128), jnp.float32),
            pltpu.VMEM((TPT, 128), jnp.float32),
            pltpu.VMEM((TPT,), jnp.int32),
            pltpu.VMEM((TPT,), jnp.int32),
            pltpu.SemaphoreType.DMA,
        ],
    )(_dispatch_body)
    xs, ws = dispatch(flat, w16, pos1, pos2)

    gwc = gate_w
    uwc = up_w
    dwc = down_w
    sgw = shared_gate_w
    suw = shared_up_w
    sdw = shared_down_w
    prefetch = jnp.concatenate([eb[:NBR, 0], nbu[:, 0]])

    shr = pl.pallas_call(
        _gemm_shared_body,
        grid=(NBS,),
        in_specs=[
            pl.BlockSpec((BLK, D), lambda bb: (bb, 0)),
            pl.BlockSpec((H, D), lambda bb: (0, 0)),
            pl.BlockSpec((H, D), lambda bb: (0, 0)),
            pl.BlockSpec((D, H), lambda bb: (0, 0)),
        ],
        out_specs=pl.BlockSpec((BLK, D), lambda bb: (bb, 0)),
        out_shape=jax.ShapeDtypeStruct((N, D), jnp.float32),
    )(flat, sgw, suw, sdw)

    HB = 2
    osr = pl.pallas_call(
        _gemm_routed_body,
        grid_spec=pltpu.PrefetchScalarGridSpec(
            num_scalar_prefetch=1,
            grid=(NBR, HB),
            in_specs=[
                pl.BlockSpec((BLK, D), lambda bb, hb, sr: (bb, 0)),
                pl.BlockSpec((BLK, 128), lambda bb, hb, sr: (bb, 0)),
                pl.BlockSpec((1, H // HB, D), lambda bb, hb, sr: (sr[bb], hb, 0)),
                pl.BlockSpec((1, H // HB, D), lambda bb, hb, sr: (sr[bb], hb, 0)),
                pl.BlockSpec((1, D, H // HB), lambda bb, hb, sr: (sr[bb], 0, hb)),
            ],
            out_specs=pl.BlockSpec((BLK, D), lambda bb, hb, sr: (bb, 0)),
        ),
        out_shape=jax.ShapeDtypeStruct((XS_ROWS, D), jnp.float32),
    )(prefetch, xs, ws, gwc, uwc, dwc)

    combine = functools.partial(
        pl.kernel,
        out_type=jax.ShapeDtypeStruct((N, D), jnp.float32),
        mesh=mesh,
        scratch_types=[
            pltpu.VMEM((_CS, D), jnp.float32),
            pltpu.VMEM((_CS, D), jnp.float32),
            pltpu.VMEM((_CS, D), jnp.float32),
            pltpu.VMEM((_CS, D), jnp.float32),
            pltpu.VMEM((_CS,), jnp.int32),
            pltpu.VMEM((_CS,), jnp.int32),
            pltpu.SemaphoreType.DMA,
        ],
    )(_combine_body)
    hidden_flat = combine(osr, shr, pos1, pos2)

    hidden = hidden_flat.reshape(b, s, d)
    aux_loss = jnp.zeros((), dtype=x.dtype)
    return hidden, aux_loss, rzl[0, 0], lb[0, 0]


# speedup vs baseline: 1.3725x; 1.3725x over previous
import functools

import jax
import jax.numpy as jnp
from jax import lax
from jax.experimental import pallas as pl
from jax.experimental.pallas import tpu as pltpu
from jax.experimental.pallas import tpu_sc as plsc

E = 8
K = 2
D = 1024
H = 2048
N = 2048
NP = N * K
BLK = 256
NBR = NP // BLK + E
NBS = N // BLK
NB = NBR + NBS
XS_ROWS = NBR * BLK
CHUNK = 128

NC, NS, L = 2, 16, 16
NW = NC * NS
TPT = N // NW


def _router_body(x_ref, rw_ref, pos_ref, w16_ref, eb_ref, nbu_ref,
                 rzl_ref, lb_ref):
    xv = x_ref[...]
    rw = rw_ref[...]
    logits = lax.dot_general(xv, rw, (((1,), (1,)), ((), ())),
                             preferred_element_type=jnp.float32)
    n = logits.shape[0]
    iota_e = lax.broadcasted_iota(jnp.int32, (n, E), 1)
    m1 = jnp.max(logits, axis=1, keepdims=True)
    i1 = jnp.min(jnp.where(logits == m1, iota_e, E), axis=1, keepdims=True)
    l2 = jnp.where(iota_e == i1, -jnp.inf, logits)
    m2 = jnp.max(l2, axis=1, keepdims=True)
    i2 = jnp.min(jnp.where(l2 == m2, iota_e, E), axis=1, keepdims=True)
    s = jnp.exp(m2 - m1)
    w1 = 1.0 / (1.0 + s)
    w2 = s * w1
    wp = jnp.concatenate([w1, w2], axis=0)
    w16_ref[...] = jnp.broadcast_to(wp, (NP, 128))
    ez = jnp.exp(logits - m1)
    denom = jnp.sum(ez, axis=1, keepdims=True)
    z = m1 + jnp.log(denom)
    rzl_ref[...] = jnp.reshape(jnp.sum(z * z) * (1e-4 / n), (1, 1))
    p_sum = jnp.sum(ez / denom, axis=0, keepdims=True)
    mask = jnp.logical_or(iota_e == i1, iota_e == i2)
    f_sum = jnp.sum(mask.astype(jnp.float32), axis=0, keepdims=True)
    lb_ref[...] = jnp.reshape(jnp.sum(p_sum * f_sum) * (float(E) / (n * n)),
                              (1, 1))

    ep = jnp.concatenate([i1, i2], axis=0)
    iota_pe = lax.broadcasted_iota(jnp.int32, (NP, E), 1)
    oh = (iota_pe == ep).astype(jnp.float32)
    counts = jnp.sum(oh, axis=0, keepdims=True)
    cp = jnp.floor((counts + (BLK - 1)) * (1.0 / BLK)) * BLK
    su = (lax.broadcasted_iota(jnp.int32, (E, E), 0)
          < lax.broadcasted_iota(jnp.int32, (E, E), 1)).astype(jnp.float32)
    offp = lax.dot_general(cp, su, (((1,), (0,)), ((), ())),
                           preferred_element_type=jnp.float32)
    nbu = jnp.sum(cp) * (1.0 / BLK)
    nbu_ref[...] = jnp.reshape(nbu, (1, 1)).astype(jnp.int32)

    tri = (lax.broadcasted_iota(jnp.int32, (CHUNK, CHUNK), 0)
           > lax.broadcasted_iota(jnp.int32, (CHUNK, CHUNK), 1)
           ).astype(jnp.float32)
    prefix = jnp.zeros((1, E), jnp.float32)
    pos_parts = []
    for c in range(NP // CHUNK):
        oh_c = oh[c * CHUNK:(c + 1) * CHUNK]
        rc = lax.dot_general(tri, oh_c, (((1,), (0,)), ((), ())),
                             preferred_element_type=jnp.float32) + prefix
        pos_c = jnp.sum(oh_c * (rc + offp), axis=1, keepdims=True)
        pos_parts.append(pos_c)
        prefix = prefix + jnp.sum(oh_c, axis=0, keepdims=True)
    pos = jnp.concatenate(pos_parts, axis=0)
    pos_ref[...] = pos.astype(jnp.int32)

    bi = lax.broadcasted_iota(jnp.int32, (NB, 1), 0).astype(jnp.float32)
    rowpos = jnp.minimum(bi * BLK, (nbu - 1.0) * BLK)
    le = (jnp.broadcast_to(offp, (NB, E)) <= rowpos).astype(jnp.float32)
    eb = jnp.sum(le, axis=1, keepdims=True) - 1.0
    eb = jnp.where(bi >= float(NBR), float(E), eb)
    eb_ref[...] = eb.astype(jnp.int32)


def _dispatch_body(x_hbm, w16_hbm, pos1_hbm, pos2_hbm, xs_hbm, ws_hbm,
                   rows_v, w1r_v, w2r_v, idx1_v, idx2_v, sem):
    wid = lax.axis_index("s") * NC + lax.axis_index("c")
    base = wid * TPT
    pltpu.sync_copy(x_hbm.at[pl.ds(base, TPT)], rows_v)
    pltpu.sync_copy(pos1_hbm.at[pl.ds(base, TPT)], idx1_v)
    pltpu.sync_copy(pos2_hbm.at[pl.ds(base, TPT)], idx2_v)
    pltpu.sync_copy(w16_hbm.at[pl.ds(base, TPT)], w1r_v)
    pltpu.sync_copy(w16_hbm.at[pl.ds(N + base, TPT)], w2r_v)
    pltpu.async_copy(rows_v, xs_hbm.at[idx1_v], sem).wait()
    pltpu.async_copy(rows_v, xs_hbm.at[idx2_v], sem).wait()
    pltpu.async_copy(w1r_v, ws_hbm.at[idx1_v], sem).wait()
    pltpu.async_copy(w2r_v, ws_hbm.at[idx2_v], sem).wait()


def _gemm_block(x_ref, gw_ref, uw_ref, dw_ref, wcol):
    xb = x_ref[...]
    gw = gw_ref[...]
    uw = uw_ref[...]
    dw = dw_ref[...]
    if gw.ndim == 3:
        gw, uw, dw = gw[0], uw[0], dw[0]
    g = lax.dot_general(xb, gw, (((1,), (1,)), ((), ())),
                        preferred_element_type=jnp.float32)
    u = lax.dot_general(xb, uw, (((1,), (1,)), ((), ())),
                        preferred_element_type=jnp.float32)
    h = jax.nn.silu(g) * u
    if wcol is not None:
        h = h * wcol
    return lax.dot_general(h, dw, (((1,), (1,)), ((), ())),
                           preferred_element_type=jnp.float32)


def _gemm_routed_body(s_ref, xs_ref, ws_ref, gw_ref, uw_ref, dw_ref,
                      os_ref):
    b = pl.program_id(0)
    nbu = s_ref[NBR]

    @pl.when(b < nbu)
    def _():
        wcol = ws_ref[...][:, 0:1]
        os_ref[...] = _gemm_block(xs_ref, gw_ref, uw_ref, dw_ref, wcol)


def _gemm_shared_body(xf_ref, gw_ref, uw_ref, dw_ref, sh_ref):
    sh_ref[...] = _gemm_block(xf_ref, gw_ref, uw_ref, dw_ref, None)


_CS = 8
_NSET = 2


def _combine_body(os_hbm, sh_hbm, pos1_hbm, pos2_hbm,
                  out_hbm, b1_v, b2_v, shb_v, outb_v, i1_v, i2_v,
                  gsems, osems):
    wid = lax.axis_index("s") * NC + lax.axis_index("c")
    base = wid * TPT
    nvec = D // L
    nch = TPT // _CS

    def start(q):
        s = q % _NSET
        b0 = base + q * _CS
        pltpu.sync_copy(pos1_hbm.at[pl.ds(b0, _CS)], i1_v[s])
        pltpu.sync_copy(pos2_hbm.at[pl.ds(b0, _CS)], i2_v[s])
        return (pltpu.async_copy(os_hbm.at[i1_v[s]], b1_v[s], gsems[s]),
                pltpu.async_copy(os_hbm.at[i2_v[s]], b2_v[s], gsems[s]),
                pltpu.async_copy(sh_hbm.at[pl.ds(b0, _CS)], shb_v[s],
                                 gsems[s]))

    pending = {0: start(0)}
    owrites = {}
    for q in range(nch):
        s = q % _NSET
        if q + 1 < nch:
            pending[q + 1] = start(q + 1)
        for cp in pending.pop(q):
            cp.wait()
        if q - _NSET in owrites:
            owrites.pop(q - _NSET).wait()
        for j in range(_CS):
            def body(cv, carry, s=s, j=j):
                sl = pl.ds(cv * L, L)
                outb_v[s][j, sl] = (b1_v[s][j, sl] + b2_v[s][j, sl]
                                    + shb_v[s][j, sl])
                return carry

            lax.fori_loop(0, nvec, body, 0)
        owrites[q] = pltpu.async_copy(
            outb_v[s], out_hbm.at[pl.ds(base + q * _CS, _CS)], osems[s])
    for cp in owrites.values():
        cp.wait()


def kernel(x, router_w, gate_w, up_w, down_w, shared_gate_w, shared_up_w,
           shared_down_w):
    b, s, d = x.shape
    flat = x.reshape(N, D)

    pos, w16, eb, nbu, rzl, lb = pl.pallas_call(
        _router_body,
        out_shape=(
            jax.ShapeDtypeStruct((NP, 1), jnp.int32),
            jax.ShapeDtypeStruct((NP, 128), jnp.float32),
            jax.ShapeDtypeStruct((NB, 1), jnp.int32),
            jax.ShapeDtypeStruct((1, 1), jnp.int32),
            jax.ShapeDtypeStruct((1, 1), jnp.float32),
            jax.ShapeDtypeStruct((1, 1), jnp.float32),
        ),
    )(flat, router_w)

    pos1 = pos[:N, 0]
    pos2 = pos[N:, 0]

    mesh = plsc.VectorSubcoreMesh(core_axis_name="c", subcore_axis_name="s",
                                  num_cores=NC, num_subcores=NS)

    dispatch = functools.partial(
        pl.kernel,
        out_type=(
            jax.ShapeDtypeStruct((XS_ROWS, D), jnp.float32),
            jax.ShapeDtypeStruct((XS_ROWS, 128), jnp.float32),
        ),
        mesh=mesh,
        scratch_types=[
            pltpu.VMEM((TPT, D), jnp.float32),
            pltpu.VMEM((TPT, 128), jnp.float32),
            pltpu.VMEM((TPT, 128), jnp.float32),
            pltpu.VMEM((TPT,), jnp.int32),
            pltpu.VMEM((TPT,), jnp.int32),
            pltpu.SemaphoreType.DMA,
        ],
    )(_dispatch_body)
    xs, ws = dispatch(flat, w16, pos1, pos2)

    gwc = gate_w
    uwc = up_w
    dwc = down_w
    sgw = shared_gate_w
    suw = shared_up_w
    sdw = shared_down_w
    prefetch = jnp.concatenate([eb[:NBR, 0], nbu[:, 0]])

    shr = pl.pallas_call(
        _gemm_shared_body,
        grid=(NBS,),
        in_specs=[
            pl.BlockSpec((BLK, D), lambda bb: (bb, 0)),
            pl.BlockSpec((H, D), lambda bb: (0, 0)),
            pl.BlockSpec((H, D), lambda bb: (0, 0)),
            pl.BlockSpec((D, H), lambda bb: (0, 0)),
        ],
        out_specs=pl.BlockSpec((BLK, D), lambda bb: (bb, 0)),
        out_shape=jax.ShapeDtypeStruct((N, D), jnp.float32),
    )(flat, sgw, suw, sdw)

    osr = pl.pallas_call(
        _gemm_routed_body,
        grid_spec=pltpu.PrefetchScalarGridSpec(
            num_scalar_prefetch=1,
            grid=(NBR,),
            in_specs=[
                pl.BlockSpec((BLK, D), lambda bb, sr: (bb, 0)),
                pl.BlockSpec((BLK, 128), lambda bb, sr: (bb, 0)),
                pl.BlockSpec((1, H, D), lambda bb, sr: (sr[bb], 0, 0)),
                pl.BlockSpec((1, H, D), lambda bb, sr: (sr[bb], 0, 0)),
                pl.BlockSpec((1, D, H), lambda bb, sr: (sr[bb], 0, 0)),
            ],
            out_specs=pl.BlockSpec((BLK, D), lambda bb, sr: (bb, 0)),
        ),
        out_shape=jax.ShapeDtypeStruct((XS_ROWS, D), jnp.float32),
    )(prefetch, xs, ws, gwc, uwc, dwc)

    combine = functools.partial(
        pl.kernel,
        out_type=jax.ShapeDtypeStruct((N, D), jnp.float32),
        mesh=mesh,
        scratch_types=[
            [pltpu.VMEM((_CS, D), jnp.float32) for _ in range(_NSET)],
            [pltpu.VMEM((_CS, D), jnp.float32) for _ in range(_NSET)],
            [pltpu.VMEM((_CS, D), jnp.float32) for _ in range(_NSET)],
            [pltpu.VMEM((_CS, D), jnp.float32) for _ in range(_NSET)],
            [pltpu.VMEM((_CS,), jnp.int32) for _ in range(_NSET)],
            [pltpu.VMEM((_CS,), jnp.int32) for _ in range(_NSET)],
            [pltpu.SemaphoreType.DMA for _ in range(_NSET)],
            [pltpu.SemaphoreType.DMA for _ in range(_NSET)],
        ],
    )(_combine_body)
    hidden_flat = combine(osr, shr, pos1, pos2)

    hidden = hidden_flat.reshape(b, s, d)
    aux_loss = jnp.zeros((), dtype=x.dtype)
    return hidden, aux_loss, rzl[0, 0], lb[0, 0]
